# baseline (speedup 1.0000x reference)
import jax
import jax.numpy as jnp
from jax import lax
from jax.experimental import pallas as pl
from jax.experimental.pallas import tpu as pltpu

_EPS = 1e-5


def _fused4d_kernel(x_ref, w_ref, b_ref, s_ref, t_ref, o_ref):
    x = x_ref[0]
    c = x.shape[0]
    pooled = jnp.sum(x, axis=(1, 2)).reshape(c, 1)
    z = jnp.dot(w_ref[...], pooled, preferred_element_type=jnp.float32)
    gate = jax.nn.sigmoid((z + b_ref[...]) * s_ref[...] + t_ref[...])
    g = (gate * s_ref[...]).reshape(c, 1, 1)
    t3 = t_ref[...].reshape(c, 1, 1)
    o_ref[0] = jnp.maximum(x * g + t3, 0.0)


def kernel(x, w, b, gamma, beta, mean, var):
    N, C, H, W = x.shape
    HW = H * W

    s_vec = gamma * lax.rsqrt(var + _EPS)
    t_vec = beta - mean * s_vec
    s_col = s_vec.reshape(C, 1)
    t_col = t_vec.reshape(C, 1)
    w_scaled = w.astype(jnp.float32) * (1.0 / HW)
    b_col = b.reshape(C, 1).astype(jnp.float32)

    out = pl.pallas_call(
        _fused4d_kernel,
        out_shape=jax.ShapeDtypeStruct((N, C, H, W), jnp.float32),
        grid=(N,),
        in_specs=[
            pl.BlockSpec((1, C, H, W), lambda i: (i, 0, 0, 0)),
            pl.BlockSpec((C, C), lambda i: (0, 0)),
            pl.BlockSpec((C, 1), lambda i: (0, 0)),
            pl.BlockSpec((C, 1), lambda i: (0, 0)),
            pl.BlockSpec((C, 1), lambda i: (0, 0)),
        ],
        out_specs=pl.BlockSpec((1, C, H, W), lambda i: (i, 0, 0, 0)),
        compiler_params=pltpu.CompilerParams(
            dimension_semantics=("arbitrary",),
            vmem_limit_bytes=56 * 1024 * 1024),
        name="fused_channel_attention_4d",
    )(x.astype(jnp.float32), w_scaled, b_col, s_col, t_col)
    return out


# speedup vs baseline: 6.2076x; 6.2076x over previous
import jax
import jax.numpy as jnp
from jax import lax
from jax.experimental import pallas as pl
from jax.experimental.pallas import tpu as pltpu

_EPS = 1e-5


def _sum_sublanes(x):
    hw = x.shape[0]
    n = hw // 128
    if n < 2:
        return jnp.sum(x, axis=0, keepdims=True)
    chunks = [x[j * 128:(j + 1) * 128] for j in range(n)]
    tail = x[n * 128:] if hw % 128 else None
    while len(chunks) > 1:
        nxt = [a + b for a, b in zip(chunks[0::2], chunks[1::2])]
        if len(chunks) % 2:
            nxt.append(chunks[-1])
        chunks = nxt
    pooled = jnp.sum(chunks[0], axis=0, keepdims=True)
    if tail is not None:
        pooled = pooled + jnp.sum(tail, axis=0, keepdims=True)
    return pooled


def _fused_nhwc_kernel(x_ref, wt_ref, b_ref, s_ref, t_ref, o_ref):
    x = x_ref[0]
    pooled = _sum_sublanes(x)

    z = jnp.dot(pooled, wt_ref[...], preferred_element_type=jnp.float32)
    gate = jax.nn.sigmoid((z + b_ref[...]) * s_ref[...] + t_ref[...])

    g = gate * s_ref[...]
    o_ref[0] = jnp.maximum(x * g + t_ref[...], 0.0)


def kernel(x, w, b, gamma, beta, mean, var):
    N, C, H, W = x.shape
    HW = H * W

    xt = x.astype(jnp.float32).transpose(0, 2, 3, 1).reshape(N, HW, C)

    s_vec = gamma * lax.rsqrt(var + _EPS)
    t_vec = beta - mean * s_vec
    s_row = s_vec.reshape(1, C)
    t_row = t_vec.reshape(1, C)
    wt = w.astype(jnp.float32).T * (1.0 / HW)
    b_row = b.reshape(1, C).astype(jnp.float32)

    out = pl.pallas_call(
        _fused_nhwc_kernel,
        out_shape=jax.ShapeDtypeStruct((N, HW, C), jnp.float32),
        grid=(N,),
        in_specs=[
            pl.BlockSpec((1, HW, C), lambda i: (i, 0, 0)),
            pl.BlockSpec((C, C), lambda i: (0, 0)),
            pl.BlockSpec((1, C), lambda i: (0, 0)),
            pl.BlockSpec((1, C), lambda i: (0, 0)),
            pl.BlockSpec((1, C), lambda i: (0, 0)),
        ],
        out_specs=pl.BlockSpec((1, HW, C), lambda i: (i, 0, 0)),
        compiler_params=pltpu.CompilerParams(
            dimension_semantics=("arbitrary",),
            vmem_limit_bytes=32 * 1024 * 1024),
        name="fused_channel_attention_nhwc",
    )(xt, wt, b_row, s_row, t_row)

    return out.reshape(N, H, W, C).transpose(0, 3, 1, 2)


# speedup vs baseline: 6.6494x; 1.0712x over previous
import jax
import jax.numpy as jnp
from jax import lax
from jax.experimental import pallas as pl
from jax.experimental.pallas import tpu as pltpu

_EPS = 1e-5


def _sum_sublanes(x):
    hw = x.shape[0]
    n = hw // 128
    if n < 2:
        return jnp.sum(x, axis=0, keepdims=True)
    chunks = [x[j * 128:(j + 1) * 128] for j in range(n)]
    tail = x[n * 128:] if hw % 128 else None
    while len(chunks) > 1:
        nxt = [a + b for a, b in zip(chunks[0::2], chunks[1::2])]
        if len(chunks) % 2:
            nxt.append(chunks[-1])
        chunks = nxt
    pooled = jnp.sum(chunks[0], axis=0, keepdims=True)
    if tail is not None:
        pooled = pooled + jnp.sum(tail, axis=0, keepdims=True)
    return pooled


def _fused_nhwc_kernel(x_ref, wt_ref, b_ref, s_ref, t_ref, o_ref):
    for k in range(x_ref.shape[0]):
        x = x_ref[k]
        pooled = _sum_sublanes(x)

        z = jnp.dot(pooled, wt_ref[...], preferred_element_type=jnp.float32)
        gate = jax.nn.sigmoid((z + b_ref[...]) * s_ref[...] + t_ref[...])

        g = gate * s_ref[...]
        o_ref[k] = jnp.maximum(x * g + t_ref[...], 0.0)


def kernel(x, w, b, gamma, beta, mean, var):
    N, C, H, W = x.shape
    HW = H * W

    xt = x.astype(jnp.float32).transpose(0, 2, 3, 1).reshape(N, HW, C)

    s_vec = gamma * lax.rsqrt(var + _EPS)
    t_vec = beta - mean * s_vec
    s_row = s_vec.reshape(1, C)
    t_row = t_vec.reshape(1, C)
    wt = w.astype(jnp.float32).T * (1.0 / HW)
    b_row = b.reshape(1, C).astype(jnp.float32)

    bn = 2 if N % 2 == 0 else 1
    out = pl.pallas_call(
        _fused_nhwc_kernel,
        out_shape=jax.ShapeDtypeStruct((N, HW, C), jnp.float32),
        grid=(N // bn,),
        in_specs=[
            pl.BlockSpec((bn, HW, C), lambda i: (i, 0, 0)),
            pl.BlockSpec((C, C), lambda i: (0, 0)),
            pl.BlockSpec((1, C), lambda i: (0, 0)),
            pl.BlockSpec((1, C), lambda i: (0, 0)),
            pl.BlockSpec((1, C), lambda i: (0, 0)),
        ],
        out_specs=pl.BlockSpec((bn, HW, C), lambda i: (i, 0, 0)),
        compiler_params=pltpu.CompilerParams(
            dimension_semantics=("arbitrary",),
            vmem_limit_bytes=56 * 1024 * 1024),
        name="fused_channel_attention_nhwc",
    )(xt, wt, b_row, s_row, t_row)

    return out.reshape(N, H, W, C).transpose(0, 3, 1, 2)


# speedup vs baseline: 6.6906x; 1.0062x over previous
import jax
import jax.numpy as jnp
from jax import lax
from jax.experimental import pallas as pl
from jax.experimental.pallas import tpu as pltpu

_EPS = 1e-5


def _sum_sublanes(x):
    hw = x.shape[0]
    n = hw // 128
    if n < 2:
        return jnp.sum(x, axis=0, keepdims=True)
    chunks = [x[j * 128:(j + 1) * 128] for j in range(n)]
    tail = x[n * 128:] if hw % 128 else None
    while len(chunks) > 1:
        nxt = [a + b for a, b in zip(chunks[0::2], chunks[1::2])]
        if len(chunks) % 2:
            nxt.append(chunks[-1])
        chunks = nxt
    pooled = jnp.sum(chunks[0], axis=0, keepdims=True)
    if tail is not None:
        pooled = pooled + jnp.sum(tail, axis=0, keepdims=True)
    return pooled


def _fused_nhwc_kernel(x_ref, wt_ref, b_ref, s_ref, t_ref, o_ref):
    for k in range(x_ref.shape[0]):
        x = x_ref[k]
        pooled = _sum_sublanes(x)

        z = jnp.dot(pooled, wt_ref[...], preferred_element_type=jnp.float32)
        gate = jax.nn.sigmoid((z + b_ref[...]) * s_ref[...] + t_ref[...])

        g = gate * s_ref[...]
        o_ref[k] = jnp.maximum(x * g + t_ref[...], 0.0)


def kernel(x, w, b, gamma, beta, mean, var):
    N, C, H, W = x.shape
    HW = H * W

    xt = x.astype(jnp.float32).transpose(0, 2, 3, 1).reshape(N, HW, C)

    s_vec = gamma * lax.rsqrt(var + _EPS)
    t_vec = beta - mean * s_vec
    s_row = s_vec.reshape(1, C)
    t_row = t_vec.reshape(1, C)
    wt = w.astype(jnp.float32).T * (1.0 / HW)
    b_row = b.reshape(1, C).astype(jnp.float32)

    bn = 4 if N % 4 == 0 else (2 if N % 2 == 0 else 1)
    out = pl.pallas_call(
        _fused_nhwc_kernel,
        out_shape=jax.ShapeDtypeStruct((N, HW, C), jnp.float32),
        grid=(N // bn,),
        in_specs=[
            pl.BlockSpec((bn, HW, C), lambda i: (i, 0, 0)),
            pl.BlockSpec((C, C), lambda i: (0, 0)),
            pl.BlockSpec((1, C), lambda i: (0, 0)),
            pl.BlockSpec((1, C), lambda i: (0, 0)),
            pl.BlockSpec((1, C), lambda i: (0, 0)),
        ],
        out_specs=pl.BlockSpec((bn, HW, C), lambda i: (i, 0, 0)),
        compiler_params=pltpu.CompilerParams(
            dimension_semantics=("arbitrary",),
            vmem_limit_bytes=56 * 1024 * 1024),
        name="fused_channel_attention_nhwc",
    )(xt, wt, b_row, s_row, t_row)

    return out.reshape(N, H, W, C).transpose(0, 3, 1, 2)
